# baseline (device time: 254882 ns/iter reference)
import jax
import jax.numpy as jnp
from jax import lax
from jax.experimental import pallas as pl
from jax.experimental.pallas import tpu as pltpu

N_DEV = 16
B = 2
SQ_LOC = 128
D_MODEL = 512
HQ_LOC = 4
DH = 64
SKV = 128
D_CHUNK = HQ_LOC * DH


def kernel(x, Wq, K_ext, V_ext, Wo):
    K_t = jnp.transpose(K_ext, (2, 0, 1, 3))
    V_t = jnp.transpose(V_ext, (2, 0, 1, 3))

    def body(x_ref, wq_ref, k_ref, v_ref, wo_ref, out_ref,
             commq, commo, sendq, recvq, sendo, recvo):
        my_pos = lax.axis_index("i")
        left = lax.rem(my_pos - 1 + N_DEV, N_DEV)
        right = lax.rem(my_pos + 1, N_DEV)

        barrier_sem = pltpu.get_barrier_semaphore()
        for nbr in (left, right):
            pl.semaphore_signal(
                barrier_sem, inc=1,
                device_id=(nbr,), device_id_type=pl.DeviceIdType.MESH,
            )
        pl.semaphore_wait(barrier_sem, 2)

        i_idx = lax.broadcasted_iota(jnp.int32, (SQ_LOC, SKV), 0)
        j_idx = lax.broadcasted_iota(jnp.int32, (SQ_LOC, SKV), 1)
        qb = my_pos * (SQ_LOC // 64) + i_idx // 64
        kb = j_idx // 64
        mask = (qb == kb) | (kb == 0) | (lax.rem(qb + kb, 3) == 0)

        def accumulate_chunk(c, wq_c, wo_c):
            kc = k_ref[pl.ds(c * HQ_LOC, HQ_LOC)]
            vc = v_ref[pl.ds(c * HQ_LOC, HQ_LOC)]
            for b in range(B):
                q_b = jnp.dot(x_ref[b], wq_c,
                              preferred_element_type=jnp.float32)
                ctx_heads = []
                for hh in range(HQ_LOC):
                    q_bh = q_b[:, hh * DH:(hh + 1) * DH]
                    k_bh = kc[hh, b]
                    v_bh = vc[hh, b]
                    scores = lax.dot_general(
                        q_bh, k_bh, (((1,), (1,)), ((), ())),
                        preferred_element_type=jnp.float32) * 0.125
                    scores = jnp.where(mask, scores, -1e9)
                    m = jnp.max(scores, axis=-1, keepdims=True)
                    w = jnp.exp(scores - m)
                    w = w / jnp.sum(w, axis=-1, keepdims=True)
                    ctx_heads.append(
                        jnp.dot(w, v_bh, preferred_element_type=jnp.float32))
                ctx_b = jnp.concatenate(ctx_heads, axis=1)
                out_ref[b] = out_ref[b] + jnp.dot(
                    ctx_b, wo_c, preferred_element_type=jnp.float32)

        out_ref[...] = jnp.zeros((B, SQ_LOC, D_MODEL), jnp.float32)

        commq[0] = wq_ref[...]
        commo[0] = wo_ref[...]
        accumulate_chunk(my_pos, commq[0], commo[0])

        def hop(h, carry):
            rq = pltpu.make_async_remote_copy(
                src_ref=commq.at[h], dst_ref=commq.at[h + 1],
                send_sem=sendq.at[h], recv_sem=recvq.at[h],
                device_id=(right,), device_id_type=pl.DeviceIdType.MESH,
            )
            ro = pltpu.make_async_remote_copy(
                src_ref=commo.at[h], dst_ref=commo.at[h + 1],
                send_sem=sendo.at[h], recv_sem=recvo.at[h],
                device_id=(right,), device_id_type=pl.DeviceIdType.MESH,
            )
            rq.start()
            ro.start()
            rq.wait()
            ro.wait()
            c = lax.rem(my_pos - h - 1 + N_DEV, N_DEV)
            accumulate_chunk(c, commq[h + 1], commo[h + 1])
            return carry

        lax.fori_loop(0, N_DEV - 1, hop, 0)

    return pl.pallas_call(
        body,
        out_shape=jax.ShapeDtypeStruct((B, SQ_LOC, D_MODEL), jnp.float32),
        in_specs=[pl.BlockSpec(memory_space=pltpu.VMEM)] * 5,
        out_specs=pl.BlockSpec(memory_space=pltpu.VMEM),
        scratch_shapes=[
            pltpu.VMEM((N_DEV, D_MODEL, D_CHUNK), jnp.float32),
            pltpu.VMEM((N_DEV, D_CHUNK, D_MODEL), jnp.float32),
            pltpu.SemaphoreType.DMA((N_DEV - 1,)),
            pltpu.SemaphoreType.DMA((N_DEV - 1,)),
            pltpu.SemaphoreType.DMA((N_DEV - 1,)),
            pltpu.SemaphoreType.DMA((N_DEV - 1,)),
        ],
        compiler_params=pltpu.CompilerParams(collective_id=0),
    )(x, Wq, K_t, V_t, Wo)


# device time: 122873 ns/iter; 2.0744x vs baseline; 2.0744x over previous
import jax
import jax.numpy as jnp
from jax import lax
from jax.experimental import pallas as pl
from jax.experimental.pallas import tpu as pltpu

N_DEV = 16
B = 2
SQ_LOC = 128
D_MODEL = 512
HQ_LOC = 4
DH = 64
SKV = 128
D_CHUNK = HQ_LOC * DH

CW_HOPS = 8
CCW_HOPS = 7
CCW_BASE = 9


def kernel(x, Wq, K_ext, V_ext, Wo):
    K_t = jnp.transpose(K_ext, (2, 0, 1, 3))
    V_t = jnp.transpose(V_ext, (2, 0, 1, 3))
    W = jnp.concatenate([Wq, Wo.T], axis=1)

    def body(x_ref, w_ref, k_ref, v_ref, out_ref,
             comm, cw_send, cw_recv, ccw_send, ccw_recv):
        my_pos = lax.axis_index("i")
        left = lax.rem(my_pos - 1 + N_DEV, N_DEV)
        right = lax.rem(my_pos + 1, N_DEV)

        barrier_sem = pltpu.get_barrier_semaphore()
        for nbr in (left, right):
            pl.semaphore_signal(
                barrier_sem, inc=1,
                device_id=(nbr,), device_id_type=pl.DeviceIdType.MESH,
            )
        pl.semaphore_wait(barrier_sem, 2)

        i_idx = lax.broadcasted_iota(jnp.int32, (SQ_LOC, SKV), 0)
        j_idx = lax.broadcasted_iota(jnp.int32, (SQ_LOC, SKV), 1)
        qb = my_pos * (SQ_LOC // 64) + i_idx // 64
        kb = j_idx // 64
        mask = (qb == kb) | (kb == 0) | (lax.rem(qb + kb, 3) == 0)

        def cw_rdma(h):
            return pltpu.make_async_remote_copy(
                src_ref=comm.at[h - 1], dst_ref=comm.at[h],
                send_sem=cw_send.at[h - 1], recv_sem=cw_recv.at[h - 1],
                device_id=(right,), device_id_type=pl.DeviceIdType.MESH,
            )

        def ccw_rdma(h):
            return pltpu.make_async_remote_copy(
                src_ref=comm.at[CCW_BASE + h - 1], dst_ref=comm.at[CCW_BASE + h],
                send_sem=ccw_send.at[h - 1], recv_sem=ccw_recv.at[h - 1],
                device_id=(left,), device_id_type=pl.DeviceIdType.MESH,
            )

        def accumulate_chunk(c, w_c):
            wq_c = w_c[:, :D_CHUNK]
            wo_t = w_c[:, D_CHUNK:]
            kc = k_ref[pl.ds(c * HQ_LOC, HQ_LOC)]
            vc = v_ref[pl.ds(c * HQ_LOC, HQ_LOC)]
            for b in range(B):
                q_b = jnp.dot(x_ref[b], wq_c,
                              preferred_element_type=jnp.float32)
                ctx_heads = []
                for hh in range(HQ_LOC):
                    q_bh = q_b[:, hh * DH:(hh + 1) * DH]
                    k_bh = kc[hh, b]
                    v_bh = vc[hh, b]
                    scores = lax.dot_general(
                        q_bh, k_bh, (((1,), (1,)), ((), ())),
                        preferred_element_type=jnp.float32) * 0.125
                    scores = jnp.where(mask, scores, -1e9)
                    m = jnp.max(scores, axis=-1, keepdims=True)
                    w = jnp.exp(scores - m)
                    w = w / jnp.sum(w, axis=-1, keepdims=True)
                    ctx_heads.append(
                        jnp.dot(w, v_bh, preferred_element_type=jnp.float32))
                ctx_b = jnp.concatenate(ctx_heads, axis=1)
                out_ref[b] = out_ref[b] + lax.dot_general(
                    ctx_b, wo_t, (((1,), (1,)), ((), ())),
                    preferred_element_type=jnp.float32)

        out_ref[...] = jnp.zeros((B, SQ_LOC, D_MODEL), jnp.float32)

        comm[0] = w_ref[...]
        comm[CCW_BASE] = w_ref[...]
        cw_rdma(1).start()
        ccw_rdma(1).start()
        accumulate_chunk(my_pos, comm[0])

        def hop(h, carry):
            cw_rdma(h).wait()

            @pl.when(h < CW_HOPS)
            def _():
                cw_rdma(h + 1).start()

            accumulate_chunk(lax.rem(my_pos - h + N_DEV, N_DEV), comm[h])

            @pl.when(h <= CCW_HOPS)
            def _():
                ccw_rdma(h).wait()

                @pl.when(h < CCW_HOPS)
                def _():
                    ccw_rdma(h + 1).start()

                accumulate_chunk(lax.rem(my_pos + h, N_DEV),
                                 comm[CCW_BASE + h])

            return carry

        lax.fori_loop(1, CW_HOPS + 1, hop, 0)

    return pl.pallas_call(
        body,
        out_shape=jax.ShapeDtypeStruct((B, SQ_LOC, D_MODEL), jnp.float32),
        in_specs=[pl.BlockSpec(memory_space=pltpu.VMEM)] * 4,
        out_specs=pl.BlockSpec(memory_space=pltpu.VMEM),
        scratch_shapes=[
            pltpu.VMEM((CCW_BASE + CCW_HOPS + 1, D_MODEL, 2 * D_CHUNK),
                       jnp.float32),
            pltpu.SemaphoreType.DMA((CW_HOPS,)),
            pltpu.SemaphoreType.DMA((CW_HOPS,)),
            pltpu.SemaphoreType.DMA((CCW_HOPS,)),
            pltpu.SemaphoreType.DMA((CCW_HOPS,)),
        ],
        compiler_params=pltpu.CompilerParams(collective_id=0),
    )(x, W, K_t, V_t)


# device time: 122763 ns/iter; 2.0762x vs baseline; 1.0009x over previous
import jax
import jax.numpy as jnp
from jax import lax
from jax.experimental import pallas as pl
from jax.experimental.pallas import tpu as pltpu

N_DEV = 16
B = 2
SQ_LOC = 128
D_MODEL = 512
HQ_LOC = 4
DH = 64
SKV = 128
D_CHUNK = HQ_LOC * DH

CW_HOPS = 8
CCW_HOPS = 7
CCW_BASE = 9


def kernel(x, Wq, K_ext, V_ext, Wo):
    K_t = jnp.transpose(K_ext, (2, 0, 1, 3))
    V_t = jnp.transpose(V_ext, (2, 0, 1, 3))
    W = jnp.concatenate([Wq, Wo.T], axis=1)

    def body(x_ref, w_ref, k_ref, v_ref, out_ref,
             comm, cw_send, cw_recv, ccw_send, ccw_recv):
        my_pos = lax.axis_index("i")
        left = lax.rem(my_pos - 1 + N_DEV, N_DEV)
        right = lax.rem(my_pos + 1, N_DEV)

        barrier_sem = pltpu.get_barrier_semaphore()
        for nbr in (left, right):
            pl.semaphore_signal(
                barrier_sem, inc=1,
                device_id=(nbr,), device_id_type=pl.DeviceIdType.MESH,
            )
        pl.semaphore_wait(barrier_sem, 2)

        i_idx = lax.broadcasted_iota(jnp.int32, (SQ_LOC, SKV), 0)
        j_idx = lax.broadcasted_iota(jnp.int32, (SQ_LOC, SKV), 1)
        qb = my_pos * (SQ_LOC // 64) + i_idx // 64
        kb = j_idx // 64
        mask = (qb == kb) | (kb == 0) | (lax.rem(qb + kb, 3) == 0)

        def cw_rdma(h):
            return pltpu.make_async_remote_copy(
                src_ref=comm.at[h - 1], dst_ref=comm.at[h],
                send_sem=cw_send.at[h - 1], recv_sem=cw_recv.at[h - 1],
                device_id=(right,), device_id_type=pl.DeviceIdType.MESH,
            )

        def ccw_rdma(h):
            return pltpu.make_async_remote_copy(
                src_ref=comm.at[CCW_BASE + h - 1], dst_ref=comm.at[CCW_BASE + h],
                send_sem=ccw_send.at[h - 1], recv_sem=ccw_recv.at[h - 1],
                device_id=(left,), device_id_type=pl.DeviceIdType.MESH,
            )

        def accumulate_chunk(c, w_c):
            wq_c = w_c[:, :D_CHUNK]
            wo_t = w_c[:, D_CHUNK:]
            kc = k_ref[pl.ds(c * HQ_LOC, HQ_LOC)]
            vc = v_ref[pl.ds(c * HQ_LOC, HQ_LOC)]
            for b in range(B):
                q_b = jnp.dot(x_ref[b], wq_c,
                              preferred_element_type=jnp.float32)
                ctx_heads = []
                for hh in range(HQ_LOC):
                    q_bh = q_b[:, hh * DH:(hh + 1) * DH]
                    k_bh = kc[hh, b]
                    v_bh = vc[hh, b]
                    scores = lax.dot_general(
                        q_bh, k_bh, (((1,), (1,)), ((), ())),
                        preferred_element_type=jnp.float32) * 0.125
                    scores = jnp.where(mask, scores, -1e9)
                    m = jnp.max(scores, axis=-1, keepdims=True)
                    w = jnp.exp(scores - m)
                    w = w / jnp.sum(w, axis=-1, keepdims=True)
                    ctx_heads.append(
                        jnp.dot(w, v_bh, preferred_element_type=jnp.float32))
                ctx_b = jnp.concatenate(ctx_heads, axis=1)
                out_ref[b] = out_ref[b] + lax.dot_general(
                    ctx_b, wo_t, (((1,), (1,)), ((), ())),
                    preferred_element_type=jnp.float32)

        out_ref[...] = jnp.zeros((B, SQ_LOC, D_MODEL), jnp.float32)

        comm[0] = w_ref[...]
        comm[CCW_BASE] = w_ref[...]
        cw_rdma(1).start()
        ccw_rdma(1).start()
        accumulate_chunk(my_pos, comm[0])

        def hop(h, carry):
            cw_rdma(h).wait()

            @pl.when(h < CW_HOPS)
            def _():
                cw_rdma(h + 1).start()

            @pl.when(h <= CCW_HOPS)
            def _():
                ccw_rdma(h).wait()

                @pl.when(h < CCW_HOPS)
                def _():
                    ccw_rdma(h + 1).start()

            accumulate_chunk(lax.rem(my_pos - h + N_DEV, N_DEV), comm[h])

            @pl.when(h <= CCW_HOPS)
            def _():
                accumulate_chunk(lax.rem(my_pos + h, N_DEV),
                                 comm[CCW_BASE + h])

            return carry

        lax.fori_loop(1, CW_HOPS + 1, hop, 0)

    return pl.pallas_call(
        body,
        out_shape=jax.ShapeDtypeStruct((B, SQ_LOC, D_MODEL), jnp.float32),
        in_specs=[pl.BlockSpec(memory_space=pltpu.VMEM)] * 4,
        out_specs=pl.BlockSpec(memory_space=pltpu.VMEM),
        scratch_shapes=[
            pltpu.VMEM((CCW_BASE + CCW_HOPS + 1, D_MODEL, 2 * D_CHUNK),
                       jnp.float32),
            pltpu.SemaphoreType.DMA((CW_HOPS,)),
            pltpu.SemaphoreType.DMA((CW_HOPS,)),
            pltpu.SemaphoreType.DMA((CCW_HOPS,)),
            pltpu.SemaphoreType.DMA((CCW_HOPS,)),
        ],
        compiler_params=pltpu.CompilerParams(collective_id=0),
    )(x, W, K_t, V_t)


# device time: 80848 ns/iter; 3.1526x vs baseline; 1.5184x over previous
import jax
import jax.numpy as jnp
from jax import lax
from jax.experimental import pallas as pl
from jax.experimental.pallas import tpu as pltpu

N_DEV = 16
B = 2
SQ_LOC = 128
D_MODEL = 512
HQ_LOC = 4
DH = 64
SKV = 128
D_CHUNK = HQ_LOC * DH

CW_HOPS = 8
CCW_HOPS = 7
CCW_BASE = 9


def kernel(x, Wq, K_ext, V_ext, Wo):
    K_t = jnp.transpose(K_ext, (2, 0, 1, 3))
    V_t = jnp.transpose(V_ext, (2, 0, 1, 3))
    W = jnp.concatenate([Wq, Wo.T], axis=1).astype(jnp.bfloat16)

    def body(x_ref, w_ref, k_ref, v_ref, out_ref,
             comm, cw_send, cw_recv, ccw_send, ccw_recv):
        my_pos = lax.axis_index("i")
        left = lax.rem(my_pos - 1 + N_DEV, N_DEV)
        right = lax.rem(my_pos + 1, N_DEV)

        barrier_sem = pltpu.get_barrier_semaphore()
        for nbr in (left, right):
            pl.semaphore_signal(
                barrier_sem, inc=1,
                device_id=(nbr,), device_id_type=pl.DeviceIdType.MESH,
            )
        pl.semaphore_wait(barrier_sem, 2)

        i_idx = lax.broadcasted_iota(jnp.int32, (SQ_LOC, SKV), 0)
        j_idx = lax.broadcasted_iota(jnp.int32, (SQ_LOC, SKV), 1)
        qb = my_pos * (SQ_LOC // 64) + i_idx // 64
        kb = j_idx // 64
        mask = (qb == kb) | (kb == 0) | (lax.rem(qb + kb, 3) == 0)

        def cw_rdma(h):
            return pltpu.make_async_remote_copy(
                src_ref=comm.at[h - 1], dst_ref=comm.at[h],
                send_sem=cw_send.at[h - 1], recv_sem=cw_recv.at[h - 1],
                device_id=(right,), device_id_type=pl.DeviceIdType.MESH,
            )

        def ccw_rdma(h):
            return pltpu.make_async_remote_copy(
                src_ref=comm.at[CCW_BASE + h - 1], dst_ref=comm.at[CCW_BASE + h],
                send_sem=ccw_send.at[h - 1], recv_sem=ccw_recv.at[h - 1],
                device_id=(left,), device_id_type=pl.DeviceIdType.MESH,
            )

        def accumulate_chunk(c, w_c):
            wq_c = w_c[:, :D_CHUNK]
            wo_t = w_c[:, D_CHUNK:]
            kc = k_ref[pl.ds(c * HQ_LOC, HQ_LOC)]
            vc = v_ref[pl.ds(c * HQ_LOC, HQ_LOC)]
            for b in range(B):
                q_b = jnp.dot(x_ref[b].astype(jnp.bfloat16), wq_c,
                              preferred_element_type=jnp.float32)
                ctx_heads = []
                for hh in range(HQ_LOC):
                    q_bh = q_b[:, hh * DH:(hh + 1) * DH]
                    k_bh = kc[hh, b]
                    v_bh = vc[hh, b]
                    scores = lax.dot_general(
                        q_bh, k_bh, (((1,), (1,)), ((), ())),
                        preferred_element_type=jnp.float32) * 0.125
                    scores = jnp.where(mask, scores, -1e9)
                    m = jnp.max(scores, axis=-1, keepdims=True)
                    w = jnp.exp(scores - m)
                    w = w / jnp.sum(w, axis=-1, keepdims=True)
                    ctx_heads.append(
                        jnp.dot(w, v_bh, preferred_element_type=jnp.float32))
                ctx_b = jnp.concatenate(ctx_heads, axis=1)
                out_ref[b] = out_ref[b] + lax.dot_general(
                    ctx_b.astype(jnp.bfloat16), wo_t, (((1,), (1,)), ((), ())),
                    preferred_element_type=jnp.float32)

        out_ref[...] = jnp.zeros((B, SQ_LOC, D_MODEL), jnp.float32)

        comm[0] = w_ref[...]
        comm[CCW_BASE] = w_ref[...]
        cw_rdma(1).start()
        ccw_rdma(1).start()
        accumulate_chunk(my_pos, comm[0])

        def hop(h, carry):
            cw_rdma(h).wait()

            @pl.when(h < CW_HOPS)
            def _():
                cw_rdma(h + 1).start()

            @pl.when(h <= CCW_HOPS)
            def _():
                ccw_rdma(h).wait()

                @pl.when(h < CCW_HOPS)
                def _():
                    ccw_rdma(h + 1).start()

            accumulate_chunk(lax.rem(my_pos - h + N_DEV, N_DEV), comm[h])

            @pl.when(h <= CCW_HOPS)
            def _():
                accumulate_chunk(lax.rem(my_pos + h, N_DEV),
                                 comm[CCW_BASE + h])

            return carry

        lax.fori_loop(1, CW_HOPS + 1, hop, 0)

    return pl.pallas_call(
        body,
        out_shape=jax.ShapeDtypeStruct((B, SQ_LOC, D_MODEL), jnp.float32),
        in_specs=[pl.BlockSpec(memory_space=pltpu.VMEM)] * 4,
        out_specs=pl.BlockSpec(memory_space=pltpu.VMEM),
        scratch_shapes=[
            pltpu.VMEM((CCW_BASE + CCW_HOPS + 1, D_MODEL, 2 * D_CHUNK),
                       jnp.bfloat16),
            pltpu.SemaphoreType.DMA((CW_HOPS,)),
            pltpu.SemaphoreType.DMA((CW_HOPS,)),
            pltpu.SemaphoreType.DMA((CCW_HOPS,)),
            pltpu.SemaphoreType.DMA((CCW_HOPS,)),
        ],
        compiler_params=pltpu.CompilerParams(collective_id=0),
    )(x, W, K_t, V_t)


# device time: 80784 ns/iter; 3.1551x vs baseline; 1.0008x over previous
import jax
import jax.numpy as jnp
from jax import lax
from jax.experimental import pallas as pl
from jax.experimental.pallas import tpu as pltpu

N_DEV = 16
B = 2
SQ_LOC = 128
D_MODEL = 512
HQ_LOC = 4
DH = 64
SKV = 128
D_CHUNK = HQ_LOC * DH

CW_HOPS = 8
CCW_HOPS = 7
CCW_BASE = 9


def kernel(x, Wq, K_ext, V_ext, Wo):
    K_t = jnp.transpose(K_ext, (2, 0, 1, 3))
    V_t = jnp.transpose(V_ext, (2, 0, 1, 3))
    W = jnp.concatenate([Wq, Wo.T], axis=1).astype(jnp.bfloat16)

    def body(x_ref, w_ref, k_ref, v_ref, out_ref,
             comm, cw_send, cw_recv, ccw_send, ccw_recv):
        my_pos = lax.axis_index("i")
        left = lax.rem(my_pos - 1 + N_DEV, N_DEV)
        right = lax.rem(my_pos + 1, N_DEV)

        barrier_sem = pltpu.get_barrier_semaphore()
        for nbr in (left, right):
            pl.semaphore_signal(
                barrier_sem, inc=1,
                device_id=(nbr,), device_id_type=pl.DeviceIdType.MESH,
            )
        pl.semaphore_wait(barrier_sem, 2)

        i_idx = lax.broadcasted_iota(jnp.int32, (SQ_LOC, SKV), 0)
        j_idx = lax.broadcasted_iota(jnp.int32, (SQ_LOC, SKV), 1)
        qb = my_pos * (SQ_LOC // 64) + i_idx // 64
        kb = j_idx // 64
        mask = (qb == kb) | (kb == 0) | (lax.rem(qb + kb, 3) == 0)
        x_bf = x_ref[...].astype(jnp.bfloat16)

        def cw_rdma(h):
            return pltpu.make_async_remote_copy(
                src_ref=comm.at[h - 1], dst_ref=comm.at[h],
                send_sem=cw_send.at[h - 1], recv_sem=cw_recv.at[h - 1],
                device_id=(right,), device_id_type=pl.DeviceIdType.MESH,
            )

        def ccw_rdma(h):
            return pltpu.make_async_remote_copy(
                src_ref=comm.at[CCW_BASE + h - 1], dst_ref=comm.at[CCW_BASE + h],
                send_sem=ccw_send.at[h - 1], recv_sem=ccw_recv.at[h - 1],
                device_id=(left,), device_id_type=pl.DeviceIdType.MESH,
            )

        def accumulate_chunk(c, w_c):
            wq_c = w_c[:, :D_CHUNK]
            wo_t = w_c[:, D_CHUNK:]
            kc = k_ref[pl.ds(c * HQ_LOC, HQ_LOC)]
            vc = v_ref[pl.ds(c * HQ_LOC, HQ_LOC)]
            for b in range(B):
                q_b = jnp.dot(x_bf[b], wq_c,
                              preferred_element_type=jnp.float32)
                ctx_heads = []
                for hh in range(HQ_LOC):
                    q_bh = q_b[:, hh * DH:(hh + 1) * DH]
                    k_bh = kc[hh, b]
                    v_bh = vc[hh, b]
                    scores = lax.dot_general(
                        q_bh, k_bh, (((1,), (1,)), ((), ())),
                        preferred_element_type=jnp.float32) * 0.125
                    scores = jnp.where(mask, scores, -1e9)
                    m = jnp.max(scores, axis=-1, keepdims=True)
                    w = jnp.exp(scores - m)
                    w = w / jnp.sum(w, axis=-1, keepdims=True)
                    ctx_heads.append(
                        jnp.dot(w, v_bh, preferred_element_type=jnp.float32))
                ctx_b = jnp.concatenate(ctx_heads, axis=1)
                out_ref[b] = out_ref[b] + lax.dot_general(
                    ctx_b.astype(jnp.bfloat16), wo_t, (((1,), (1,)), ((), ())),
                    preferred_element_type=jnp.float32)

        out_ref[...] = jnp.zeros((B, SQ_LOC, D_MODEL), jnp.float32)

        comm[0] = w_ref[...]
        comm[CCW_BASE] = w_ref[...]
        cw_rdma(1).start()
        ccw_rdma(1).start()
        accumulate_chunk(my_pos, comm[0])

        def hop(h, carry):
            cw_rdma(h).wait()

            @pl.when(h < CW_HOPS)
            def _():
                cw_rdma(h + 1).start()

            @pl.when(h <= CCW_HOPS)
            def _():
                ccw_rdma(h).wait()

                @pl.when(h < CCW_HOPS)
                def _():
                    ccw_rdma(h + 1).start()

            accumulate_chunk(lax.rem(my_pos - h + N_DEV, N_DEV), comm[h])

            @pl.when(h <= CCW_HOPS)
            def _():
                accumulate_chunk(lax.rem(my_pos + h, N_DEV),
                                 comm[CCW_BASE + h])

            return carry

        lax.fori_loop(1, CW_HOPS + 1, hop, 0)

    return pl.pallas_call(
        body,
        out_shape=jax.ShapeDtypeStruct((B, SQ_LOC, D_MODEL), jnp.float32),
        in_specs=[pl.BlockSpec(memory_space=pltpu.VMEM)] * 4,
        out_specs=pl.BlockSpec(memory_space=pltpu.VMEM),
        scratch_shapes=[
            pltpu.VMEM((CCW_BASE + CCW_HOPS + 1, D_MODEL, 2 * D_CHUNK),
                       jnp.bfloat16),
            pltpu.SemaphoreType.DMA((CW_HOPS,)),
            pltpu.SemaphoreType.DMA((CW_HOPS,)),
            pltpu.SemaphoreType.DMA((CCW_HOPS,)),
            pltpu.SemaphoreType.DMA((CCW_HOPS,)),
        ],
        compiler_params=pltpu.CompilerParams(collective_id=0),
    )(x, W, K_t, V_t)
